# baseline (speedup 1.0000x reference)
import functools

import jax
import jax.numpy as jnp
from jax import lax
from jax.experimental import pallas as pl
from jax.experimental.pallas import tpu as pltpu
from jax.experimental.pallas import tpu_sc as plsc

VOCAB = 1000000
EMBED = 16
BATCH = 16384
HIST = 200

NC = 2
NS = 16
L = 16
NW = NC * NS
B_PER_W = BATCH // NW
N_GROUPS = B_PER_W // L
G_IDX = L * HIST

_TV_BLOCK = 8192


def _tv_body(tbl_ref, v_ref, out_ref):
  out_ref[...] = jnp.sum(tbl_ref[...] * v_ref[...], axis=1)


def _table_matvec(table, v):
  return pl.pallas_call(
      _tv_body,
      grid=(pl.cdiv(VOCAB, _TV_BLOCK),),
      in_specs=[
          pl.BlockSpec((_TV_BLOCK, EMBED), lambda i: (i, 0)),
          pl.BlockSpec((1, EMBED), lambda i: (0, 0)),
      ],
      out_specs=pl.BlockSpec((_TV_BLOCK,), lambda i: (i,)),
      out_shape=jax.ShapeDtypeStruct((VOCAB,), jnp.float32),
  )(table, v)


def _make_sc_kernel():
  mesh = plsc.VectorSubcoreMesh(core_axis_name="c", subcore_axis_name="s")

  @functools.partial(
      pl.kernel,
      mesh=mesh,
      compiler_params=pltpu.CompilerParams(use_tc_tiling_on_sc=False),
      out_type=jax.ShapeDtypeStruct((BATCH,), jnp.float32),
      scratch_types=[
          pltpu.VMEM((G_IDX,), jnp.int32),
          pltpu.VMEM((G_IDX,), jnp.int32),
          pltpu.VMEM((G_IDX,), jnp.float32),
          pltpu.VMEM((L,), jnp.float32),
          pltpu.VMEM((B_PER_W,), jnp.float32),
          pltpu.SemaphoreType.DMA,
          pltpu.SemaphoreType.DMA,
      ],
  )
  def sc_pool(xflat, tvals, cc, out, idx_v, xt_v, vals_v, cc_v, out_v,
              sem1, sem2):
    wid = lax.axis_index("s") * NC + lax.axis_index("c")
    base = wid * B_PER_W
    pltpu.sync_copy(cc, cc_v)
    ccvec = cc_v[...]
    lane = lax.iota(jnp.int32, L)
    inv = jnp.float32(1.0 / HIST)

    def g_body(gi, carry):
      rowstart = (base + gi * L + lane) * HIST
      for li in range(HIST):
        idx_v[pl.ds(li * L, L)] = rowstart + li
      pltpu.async_copy(xflat.at[idx_v], xt_v, sem1).wait()
      pltpu.async_copy(tvals.at[xt_v], vals_v, sem2).wait()
      accs = [jnp.zeros((L,), jnp.float32) for _ in range(8)]
      for li in range(HIST):
        accs[li % 8] = accs[li % 8] + vals_v[pl.ds(li * L, L)]
      acc = (((accs[0] + accs[1]) + (accs[2] + accs[3]))
             + ((accs[4] + accs[5]) + (accs[6] + accs[7])))
      z = acc * inv + ccvec
      out_v[pl.ds(gi * L, L)] = 1.0 / (1.0 + jnp.exp(-z))
      return carry

    lax.fori_loop(0, N_GROUPS, g_body, 0)
    pltpu.sync_copy(out_v, out.at[pl.ds(base, B_PER_W)])

  return sc_pool


_SC_POOL = _make_sc_kernel()


def kernel(x, table, W1, b1, W2, b2):
  v = (W2 @ W1).reshape(1, EMBED).astype(jnp.float32)
  c = (W2 @ b1 + b2).reshape(())
  cc = jnp.full((L,), c, jnp.float32)
  tvals = _table_matvec(table, v)
  out = _SC_POOL(x.reshape(-1), tvals, cc)
  return out.reshape(BATCH, 1)


# speedup vs baseline: 1.0034x; 1.0034x over previous
import functools

import jax
import jax.numpy as jnp
from jax import lax
from jax.experimental import pallas as pl
from jax.experimental.pallas import tpu as pltpu
from jax.experimental.pallas import tpu_sc as plsc

VOCAB = 1000000
EMBED = 16
BATCH = 16384
HIST = 200

NC = 2
NS = 16
L = 16
NW = NC * NS
B_PER_W = BATCH // NW
N_GROUPS = B_PER_W // L
G_IDX = L * HIST

_TV_ROWS = 2048
_TV_GRID = pl.cdiv(VOCAB // 8, _TV_ROWS)
_TO_ROWS = _TV_ROWS // 16
_T_ROWS = _TV_GRID * _TO_ROWS
_T_PAD = _T_ROWS * 128


def _tv_body(tbl_ref, vt_ref, out_ref):
  prod = tbl_ref[...] * vt_ref[...]
  out_ref[...] = jnp.sum(prod.reshape(_TV_ROWS, 8, EMBED), axis=2).reshape(
      _TO_ROWS, 128)


def _table_matvec(table128, vtile):
  return pl.pallas_call(
      _tv_body,
      grid=(_TV_GRID,),
      in_specs=[
          pl.BlockSpec((_TV_ROWS, 128), lambda i: (i, 0)),
          pl.BlockSpec((1, 128), lambda i: (0, 0)),
      ],
      out_specs=pl.BlockSpec((_TO_ROWS, 128), lambda i: (i, 0)),
      out_shape=jax.ShapeDtypeStruct((_T_ROWS, 128), jnp.float32),
  )(table128, vtile)


def _make_sc_kernel():
  mesh = plsc.VectorSubcoreMesh(core_axis_name="c", subcore_axis_name="s")

  @functools.partial(
      pl.kernel,
      mesh=mesh,
      compiler_params=pltpu.CompilerParams(use_tc_tiling_on_sc=False),
      out_type=jax.ShapeDtypeStruct((BATCH,), jnp.float32),
      scratch_types=[
          pltpu.VMEM((G_IDX,), jnp.int32),
          pltpu.VMEM((G_IDX,), jnp.int32),
          pltpu.VMEM((G_IDX,), jnp.float32),
          pltpu.VMEM((L,), jnp.float32),
          pltpu.VMEM((B_PER_W,), jnp.float32),
          pltpu.SemaphoreType.DMA,
          pltpu.SemaphoreType.DMA,
      ],
  )
  def sc_pool(xflat, tvals, cc, out, idx_v, xt_v, vals_v, cc_v, out_v,
              sem1, sem2):
    wid = lax.axis_index("s") * NC + lax.axis_index("c")
    base = wid * B_PER_W
    pltpu.sync_copy(cc, cc_v)
    ccvec = cc_v[...]
    lane = lax.iota(jnp.int32, L)
    inv = jnp.float32(1.0 / HIST)

    def g_body(gi, carry):
      rowstart = (base + gi * L + lane) * HIST
      for li in range(HIST):
        idx_v[pl.ds(li * L, L)] = rowstart + li
      pltpu.async_copy(xflat.at[idx_v], xt_v, sem1).wait()
      pltpu.async_copy(tvals.at[xt_v], vals_v, sem2).wait()
      accs = [jnp.zeros((L,), jnp.float32) for _ in range(8)]
      for li in range(HIST):
        accs[li % 8] = accs[li % 8] + vals_v[pl.ds(li * L, L)]
      acc = (((accs[0] + accs[1]) + (accs[2] + accs[3]))
             + ((accs[4] + accs[5]) + (accs[6] + accs[7])))
      z = acc * inv + ccvec
      out_v[pl.ds(gi * L, L)] = 1.0 / (1.0 + jnp.exp(-z))
      return carry

    lax.fori_loop(0, N_GROUPS, g_body, 0)
    pltpu.sync_copy(out_v, out.at[pl.ds(base, B_PER_W)])

  return sc_pool


_SC_POOL = _make_sc_kernel()


def kernel(x, table, W1, b1, W2, b2):
  v = (W2 @ W1).reshape(EMBED).astype(jnp.float32)
  c = (W2 @ b1 + b2).reshape(())
  cc = jnp.full((L,), c, jnp.float32)
  vtile = jnp.tile(v, 8).reshape(1, 128)
  tvals = _table_matvec(table.reshape(VOCAB // 8, 128), vtile).reshape(-1)
  out = _SC_POOL(x.reshape(-1), tvals, cc)
  return out.reshape(BATCH, 1)


# speedup vs baseline: 1.6326x; 1.6270x over previous
import functools

import jax
import jax.numpy as jnp
from jax import lax
from jax.experimental import pallas as pl
from jax.experimental.pallas import tpu as pltpu
from jax.experimental.pallas import tpu_sc as plsc

VOCAB = 1000000
EMBED = 16
BATCH = 16384
HIST = 200

NC = 2
NS = 16
L = 16
NW = NC * NS
B_PER_W = BATCH // NW
CHUNK_ROWS = 16
CHUNK_IDX = CHUNK_ROWS * HIST
N_CHUNKS = B_PER_W // CHUNK_ROWS


def _make_sc_kernel():
  mesh = plsc.VectorSubcoreMesh(core_axis_name="c", subcore_axis_name="s")

  @functools.partial(
      pl.kernel,
      mesh=mesh,
      compiler_params=pltpu.CompilerParams(use_tc_tiling_on_sc=False),
      out_type=jax.ShapeDtypeStruct((BATCH, EMBED), jnp.float32),
      scratch_types=[
          pltpu.VMEM((CHUNK_IDX,), jnp.int32),
          pltpu.VMEM((CHUNK_IDX,), jnp.int32),
          pltpu.VMEM((CHUNK_IDX, EMBED), jnp.float32),
          pltpu.VMEM((CHUNK_IDX, EMBED), jnp.float32),
          pltpu.VMEM((B_PER_W, EMBED), jnp.float32),
          pltpu.SemaphoreType.DMA,
          pltpu.SemaphoreType.DMA,
          pltpu.SemaphoreType.DMA,
          pltpu.SemaphoreType.DMA,
      ],
  )
  def sc_embed_sum(xflat, table, out, idx0, idx1, buf0, buf1, sums_v,
                   si0, si1, sg0, sg1):
    wid = lax.axis_index("s") * NC + lax.axis_index("c")
    base = wid * B_PER_W
    idxb = (idx0, idx1)
    bufb = (buf0, buf1)
    sib = (si0, si1)
    sgb = (sg0, sg1)

    def idx_start(ci, b):
      off = (base + ci * CHUNK_ROWS) * HIST
      pltpu.async_copy(xflat.at[pl.ds(off, CHUNK_IDX)], idxb[b], sib[b])

    def idx_wait(b):
      pltpu.make_async_copy(xflat.at[pl.ds(0, CHUNK_IDX)], idxb[b],
                            sib[b]).wait()

    def g_start(b):
      pltpu.async_copy(table.at[idxb[b]], bufb[b], sgb[b])

    def g_wait(b):
      pltpu.make_async_copy(table.at[idxb[b]], bufb[b], sgb[b]).wait()

    def pool(ci, b):
      buf = bufb[b]
      for r in range(CHUNK_ROWS):
        def l_body(li, accs, r=r):
          p = r * HIST + li * 8
          return tuple(accs[j] + buf[p + j] for j in range(8))
        accs = lax.fori_loop(0, HIST // 8, l_body,
                             tuple(jnp.zeros((L,), jnp.float32)
                                   for _ in range(8)))
        acc = (((accs[0] + accs[1]) + (accs[2] + accs[3]))
               + ((accs[4] + accs[5]) + (accs[6] + accs[7])))
        sums_v[ci * CHUNK_ROWS + r] = acc

    idx_start(0, 0)
    idx_wait(0)
    g_start(0)
    idx_start(1, 1)

    def pair_body(cp, carry):
      for half in (0, 1):
        ci = cp * 2 + half
        b = half
        nb = 1 - half
        g_wait(b)

        @pl.when(ci + 1 < N_CHUNKS)
        def _():
          idx_wait(nb)
          g_start(nb)

        @pl.when(ci + 2 < N_CHUNKS)
        def _():
          idx_start(ci + 2, b)

        pool(ci, b)
      return carry

    lax.fori_loop(0, N_CHUNKS // 2, pair_body, 0)
    pltpu.sync_copy(sums_v, out.at[pl.ds(base, B_PER_W), :])

  return sc_embed_sum


_SC_EMBED_SUM = _make_sc_kernel()

_TC_BLOCK = 4096


def _tc_head_body(sums_ref, v_ref, c_ref, out_ref):
  z = jnp.sum(sums_ref[...] * v_ref[...], axis=1, keepdims=True)
  z = z * jnp.float32(1.0 / HIST) + c_ref[0, 0]
  out_ref[...] = 1.0 / (1.0 + jnp.exp(-z))


def _tc_head(sums, v, c):
  grid = BATCH // _TC_BLOCK
  return pl.pallas_call(
      _tc_head_body,
      grid=(grid,),
      in_specs=[
          pl.BlockSpec((_TC_BLOCK, EMBED), lambda i: (i, 0)),
          pl.BlockSpec((1, EMBED), lambda i: (0, 0)),
          pl.BlockSpec(memory_space=pltpu.SMEM),
      ],
      out_specs=pl.BlockSpec((_TC_BLOCK, 1), lambda i: (i, 0)),
      out_shape=jax.ShapeDtypeStruct((BATCH, 1), jnp.float32),
  )(sums, v, c)


def kernel(x, table, W1, b1, W2, b2):
  v = (W2 @ W1).reshape(1, EMBED).astype(jnp.float32)
  c = (W2 @ b1 + b2).reshape(1, 1)
  sums = _SC_EMBED_SUM(x.reshape(-1), table)
  return _tc_head(sums, v, c.astype(jnp.float32))


# speedup vs baseline: 5.1102x; 3.1301x over previous
import functools

import jax
import jax.numpy as jnp
from jax import lax
from jax.experimental import pallas as pl
from jax.experimental.pallas import tpu as pltpu
from jax.experimental.pallas import tpu_sc as plsc

VOCAB = 1000000
EMBED = 16
BATCH = 16384
HIST = 200

NC = 2
NS = 16
L = 16
NW = NC * NS
B_PER_W = BATCH // NW
NACC = B_PER_W // L
TILE_IDX = B_PER_W * HIST

_TV_COLS = 65536


def _tv_body(tbl_ref, v_ref, out_ref):
  out_ref[...] = jnp.sum(tbl_ref[...] * v_ref[...], axis=0)


def _table_matvec(table_t, v):
  return pl.pallas_call(
      _tv_body,
      grid=(pl.cdiv(VOCAB, _TV_COLS),),
      in_specs=[
          pl.BlockSpec((EMBED, _TV_COLS), lambda i: (0, i)),
          pl.BlockSpec((EMBED, 1), lambda i: (0, 0)),
      ],
      out_specs=pl.BlockSpec((_TV_COLS,), lambda i: (i,)),
      out_shape=jax.ShapeDtypeStruct((VOCAB,), jnp.float32),
  )(table_t, v)


def _make_sc_kernel():
  mesh = plsc.VectorSubcoreMesh(core_axis_name="c", subcore_axis_name="s")

  @functools.partial(
      pl.kernel,
      mesh=mesh,
      compiler_params=pltpu.CompilerParams(use_tc_tiling_on_sc=False),
      out_type=jax.ShapeDtypeStruct((BATCH,), jnp.float32),
      scratch_types=[
          pltpu.VMEM((TILE_IDX,), jnp.int32),
          pltpu.VMEM((B_PER_W,), jnp.float32),
          pltpu.VMEM((B_PER_W,), jnp.float32),
          pltpu.VMEM((L,), jnp.float32),
          pltpu.VMEM((B_PER_W,), jnp.float32),
          pltpu.SemaphoreType.DMA,
          pltpu.SemaphoreType.DMA,
          pltpu.SemaphoreType.DMA,
      ],
  )
  def sc_pool(xtflat, tvals, cc, out, xtile_v, val0, val1, cc_v, out_v,
              ss, sg0, sg1):
    wid = lax.axis_index("s") * NC + lax.axis_index("c")
    col0 = wid * B_PER_W
    valb = (val0, val1)
    sgb = (sg0, sg1)
    pltpu.sync_copy(cc, cc_v)
    ccvec = cc_v[...]
    inv = jnp.float32(1.0 / HIST)

    def stage_body(li, carry):
      pltpu.async_copy(xtflat.at[pl.ds(li * BATCH + col0, B_PER_W)],
                       xtile_v.at[pl.ds(li * B_PER_W, B_PER_W)], ss)
      return carry

    lax.fori_loop(0, HIST, stage_body, 0)
    pltpu.make_async_copy(xtflat.at[pl.ds(0, TILE_IDX)], xtile_v, ss).wait()

    def g_start(li, b):
      pltpu.async_copy(tvals.at[xtile_v.at[pl.ds(li * B_PER_W, B_PER_W)]],
                       valb[b], sgb[b])

    def g_wait(b):
      pltpu.make_async_copy(tvals.at[xtile_v.at[pl.ds(0, B_PER_W)]],
                            valb[b], sgb[b]).wait()

    g_start(0, 0)
    g_start(1, 1)

    def pair_body(lp, accs):
      for half in (0, 1):
        li = lp * 2 + half
        g_wait(half)

        @pl.when(li + 2 < HIST)
        def _():
          g_start(li + 2, half)

        vv = valb[half]
        accs = tuple(accs[j] + vv[pl.ds(j * L, L)] for j in range(NACC))
      return accs

    zero = jnp.zeros((L,), jnp.float32)
    accs = lax.fori_loop(0, HIST // 2, pair_body, (zero,) * NACC)
    for j in range(NACC):
      z = accs[j] * inv + ccvec
      out_v[pl.ds(j * L, L)] = 1.0 / (1.0 + jnp.exp(-z))
    pltpu.sync_copy(out_v, out.at[pl.ds(col0, B_PER_W)])

  return sc_pool


_SC_POOL = _make_sc_kernel()


def kernel(x, table, W1, b1, W2, b2):
  v = (W2 @ W1).reshape(EMBED, 1).astype(jnp.float32)
  c = (W2 @ b1 + b2).reshape(())
  cc = jnp.full((L,), c, jnp.float32)
  tvals = _table_matvec(table.T, v)
  out = _SC_POOL(x.T.reshape(-1), tvals, cc)
  return out.reshape(BATCH, 1)


# speedup vs baseline: 6.1853x; 1.2104x over previous
import functools

import jax
import jax.numpy as jnp
from jax import lax
from jax.experimental import pallas as pl
from jax.experimental.pallas import tpu as pltpu
from jax.experimental.pallas import tpu_sc as plsc

VOCAB = 1000000
EMBED = 16
BATCH = 16384
HIST = 200

NC = 2
NS = 16
L = 16
NW = NC * NS
B_PER_W = BATCH // NW
NACC = B_PER_W // L
TILE_IDX = B_PER_W * HIST

_TV_COLS = 65536


def _tv_body(tbl_ref, v_ref, out_ref):
  out_ref[...] = jnp.sum(tbl_ref[...] * v_ref[...], axis=0)


def _table_matvec(table_t, v):
  return pl.pallas_call(
      _tv_body,
      grid=(pl.cdiv(VOCAB, _TV_COLS),),
      in_specs=[
          pl.BlockSpec((EMBED, _TV_COLS), lambda i: (0, i)),
          pl.BlockSpec((EMBED, 1), lambda i: (0, 0)),
      ],
      out_specs=pl.BlockSpec((_TV_COLS,), lambda i: (i,)),
      out_shape=jax.ShapeDtypeStruct((VOCAB,), jnp.float32),
  )(table_t, v)


def _make_sc_kernel():
  mesh = plsc.VectorSubcoreMesh(core_axis_name="c", subcore_axis_name="s")

  @functools.partial(
      pl.kernel,
      mesh=mesh,
      compiler_params=pltpu.CompilerParams(use_tc_tiling_on_sc=False),
      out_type=jax.ShapeDtypeStruct((BATCH,), jnp.float32),
      scratch_types=[
          pltpu.VMEM_SHARED((VOCAB,), jnp.float32),
          pltpu.VMEM((B_PER_W,), jnp.int32),
          pltpu.VMEM((B_PER_W,), jnp.int32),
          pltpu.VMEM((B_PER_W,), jnp.float32),
          pltpu.VMEM((B_PER_W,), jnp.float32),
          pltpu.VMEM((L,), jnp.float32),
          pltpu.VMEM((B_PER_W,), jnp.float32),
          pltpu.SemaphoreType.DMA,
          pltpu.SemaphoreType.DMA,
          pltpu.SemaphoreType.DMA,
          pltpu.SemaphoreType.DMA,
      ],
  )
  def sc_pool(xtflat, tvals, cc, out, shared_t, xs0, xs1, val0, val1, cc_v,
              out_v, sx0, sx1, sg0, sg1):
    sid = lax.axis_index("s")
    wid = sid * NC + lax.axis_index("c")
    col0 = wid * B_PER_W
    xsb = (xs0, xs1)
    sxb = (sx0, sx1)
    valb = (val0, val1)
    sgb = (sg0, sg1)
    pltpu.sync_copy(cc, cc_v)
    ccvec = cc_v[...]
    inv = jnp.float32(1.0 / HIST)

    def stage(li, b):
      pltpu.async_copy(xtflat.at[pl.ds(li * BATCH + col0, B_PER_W)],
                       xsb[b], sxb[b])

    def stage_wait(b):
      pltpu.make_async_copy(xtflat.at[pl.ds(0, B_PER_W)], xsb[b],
                            sxb[b]).wait()

    def g_start(b):
      pltpu.async_copy(shared_t.at[xsb[b]], valb[b], sgb[b])

    def g_wait(b):
      pltpu.make_async_copy(shared_t.at[xsb[b]], valb[b], sgb[b]).wait()

    stage(0, 0)
    stage(1, 1)

    @pl.when(sid == 0)
    def _():
      pltpu.sync_copy(tvals, shared_t)

    plsc.subcore_barrier()
    stage_wait(0)
    g_start(0)

    def pair_body(lp, accs):
      for half in (0, 1):
        li = lp * 2 + half
        b = half
        nb = 1 - half

        @pl.when(li + 1 < HIST)
        def _():
          stage_wait(nb)
          g_start(nb)

        g_wait(b)

        @pl.when(li + 2 < HIST)
        def _():
          stage(li + 2, b)

        vv = valb[b]
        accs = tuple(accs[j] + vv[pl.ds(j * L, L)] for j in range(NACC))
      return accs

    zero = jnp.zeros((L,), jnp.float32)
    accs = lax.fori_loop(0, HIST // 2, pair_body, (zero,) * NACC)
    for j in range(NACC):
      z = accs[j] * inv + ccvec
      out_v[pl.ds(j * L, L)] = 1.0 / (1.0 + jnp.exp(-z))
    pltpu.sync_copy(out_v, out.at[pl.ds(col0, B_PER_W)])

  return sc_pool


_SC_POOL = _make_sc_kernel()


def kernel(x, table, W1, b1, W2, b2):
  v = (W2 @ W1).reshape(EMBED, 1).astype(jnp.float32)
  c = (W2 @ b1 + b2).reshape(())
  cc = jnp.full((L,), c, jnp.float32)
  tvals = _table_matvec(table.T, v)
  out = _SC_POOL(x.T.reshape(-1), tvals, cc)
  return out.reshape(BATCH, 1)


# speedup vs baseline: 10.5588x; 1.7071x over previous
import functools

import jax
import jax.numpy as jnp
from jax import lax
from jax.experimental import pallas as pl
from jax.experimental.pallas import tpu as pltpu
from jax.experimental.pallas import tpu_sc as plsc

VOCAB = 1000000
EMBED = 16
BATCH = 16384
HIST = 200

NC = 2
NS = 16
L = 16
NW = NC * NS
B_PER_W = BATCH // NW
NACC = B_PER_W // L
SLAB_L = 4
SLAB = SLAB_L * B_PER_W
N_IT = HIST // SLAB_L

_TV_COLS = 65536


def _tv_body(tbl_ref, v_ref, out_ref):
  out_ref[...] = jnp.sum(tbl_ref[...] * v_ref[...], axis=0)


def _table_matvec(table_t, v):
  return pl.pallas_call(
      _tv_body,
      grid=(pl.cdiv(VOCAB, _TV_COLS),),
      in_specs=[
          pl.BlockSpec((EMBED, _TV_COLS), lambda i: (0, i)),
          pl.BlockSpec((EMBED, 1), lambda i: (0, 0)),
      ],
      out_specs=pl.BlockSpec((_TV_COLS,), lambda i: (i,)),
      out_shape=jax.ShapeDtypeStruct((VOCAB,), jnp.float32),
  )(table_t, v)


def _make_sc_kernel():
  mesh = plsc.VectorSubcoreMesh(core_axis_name="c", subcore_axis_name="s")

  @functools.partial(
      pl.kernel,
      mesh=mesh,
      compiler_params=pltpu.CompilerParams(use_tc_tiling_on_sc=False),
      out_type=jax.ShapeDtypeStruct((BATCH,), jnp.float32),
      scratch_types=[
          pltpu.VMEM_SHARED((VOCAB,), jnp.float32),
          pltpu.VMEM((SLAB,), jnp.int32),
          pltpu.VMEM((SLAB,), jnp.int32),
          pltpu.VMEM((SLAB,), jnp.float32),
          pltpu.VMEM((SLAB,), jnp.float32),
          pltpu.VMEM((L,), jnp.float32),
          pltpu.VMEM((B_PER_W,), jnp.float32),
          pltpu.SemaphoreType.DMA,
          pltpu.SemaphoreType.DMA,
          pltpu.SemaphoreType.DMA,
          pltpu.SemaphoreType.DMA,
      ],
  )
  def sc_pool(xtflat, tvals, cc, out, shared_t, xs0, xs1, val0, val1, cc_v,
              out_v, sx0, sx1, sg0, sg1):
    sid = lax.axis_index("s")
    wid = sid * NC + lax.axis_index("c")
    col0 = wid * B_PER_W
    xsb = (xs0, xs1)
    sxb = (sx0, sx1)
    valb = (val0, val1)
    sgb = (sg0, sg1)
    pltpu.sync_copy(cc, cc_v)
    ccvec = cc_v[...]
    inv = jnp.float32(1.0 / HIST)

    def stage(it, b):
      for q in range(SLAB_L):
        pltpu.async_copy(
            xtflat.at[pl.ds((it * SLAB_L + q) * BATCH + col0, B_PER_W)],
            xsb[b].at[pl.ds(q * B_PER_W, B_PER_W)], sxb[b])

    def stage_wait(b):
      pltpu.make_async_copy(xtflat.at[pl.ds(0, SLAB)], xsb[b],
                            sxb[b]).wait()

    def g_start(b):
      pltpu.async_copy(shared_t.at[xsb[b]], valb[b], sgb[b])

    def g_wait(b):
      pltpu.make_async_copy(shared_t.at[xsb[b]], valb[b], sgb[b]).wait()

    stage(0, 0)
    stage(1, 1)

    @pl.when(sid == 0)
    def _():
      pltpu.sync_copy(tvals, shared_t)

    plsc.subcore_barrier()
    stage_wait(0)
    g_start(0)

    def pair_body(lp, accs):
      for half in (0, 1):
        it = lp * 2 + half
        b = half
        nb = 1 - half

        @pl.when(it + 1 < N_IT)
        def _():
          stage_wait(nb)
          g_start(nb)

        g_wait(b)

        @pl.when(it + 2 < N_IT)
        def _():
          stage(it + 2, b)

        vv = valb[b]
        for q in range(SLAB_L):
          accs = tuple(accs[j] + vv[pl.ds((q * NACC + j) * L, L)]
                       for j in range(NACC))
      return accs

    zero = jnp.zeros((L,), jnp.float32)
    accs = lax.fori_loop(0, N_IT // 2, pair_body, (zero,) * NACC)
    for j in range(NACC):
      z = accs[j] * inv + ccvec
      out_v[pl.ds(j * L, L)] = 1.0 / (1.0 + jnp.exp(-z))
    pltpu.sync_copy(out_v, out.at[pl.ds(col0, B_PER_W)])

  return sc_pool


_SC_POOL = _make_sc_kernel()


def kernel(x, table, W1, b1, W2, b2):
  v = (W2 @ W1).reshape(EMBED, 1).astype(jnp.float32)
  c = (W2 @ b1 + b2).reshape(())
  cc = jnp.full((L,), c, jnp.float32)
  tvals = _table_matvec(table.T, v)
  out = _SC_POOL(x.T.reshape(-1), tvals, cc)
  return out.reshape(BATCH, 1)
